# baseline (speedup 1.0000x reference)
import jax
import jax.numpy as jnp
from jax.experimental import pallas as pl
from jax.experimental.pallas import tpu as pltpu

_EPS = 1e-8
_B = 4096
_D = 256
_BLK = 1024
_R = _B // _BLK


def _normalize(x):
    nrm2 = jax.lax.dot_general(
        x * x, jnp.ones((_D, 1), jnp.float32), (((1,), (0,)), ((), ())),
        preferred_element_type=jnp.float32)
    inv = 1.0 / jnp.maximum(jnp.sqrt(nrm2), _EPS)
    return (x * inv).astype(jnp.bfloat16)


def _dot_nt(a, b):
    return jax.lax.dot_general(
        a, b, (((1,), (1,)), ((), ())),
        preferred_element_type=jnp.float32)


def _mask_diag(d):
    diag = (jax.lax.broadcasted_iota(jnp.int32, (_BLK, _BLK), 0)
            == jax.lax.broadcasted_iota(jnp.int32, (_BLK, _BLK), 1))
    return jnp.where(diag, jnp.float32(-1.0), d)


def _rowmax(d):
    parts = [d[:, c * 128:(c + 1) * 128] for c in range(_BLK // 128)]
    while len(parts) > 1:
        parts = [jnp.maximum(parts[p], parts[p + 1])
                 for p in range(0, len(parts), 2)]
    return jnp.max(parts[0], axis=1, keepdims=True)


def _logdist_sum(m):
    dist2 = 2.0 + _D * _EPS * _EPS - 2.0 * m
    dist = jnp.sqrt(jnp.maximum(dist2, 0.0))
    return jnp.sum(jnp.log(dist + _EPS))


def _koleo_body(x_ref, acc_ref, y_ref, macc_ref):
    i = pl.program_id(0)

    xn = _normalize(x_ref[...])
    y_ref[pl.ds(i * _BLK, _BLK), :] = xn
    dii = _mask_diag(_dot_nt(xn, xn))
    macc_ref[pl.ds(i * _BLK, _BLK), :] = _rowmax(dii)

    for kk in range(_R - 1):
        @pl.when(kk < i)
        def _pair(kk=kk):
            yk = y_ref[kk * _BLK:(kk + 1) * _BLK, :]
            dik = _dot_nt(xn, yk)
            macc_ref[pl.ds(i * _BLK, _BLK), :] = jnp.maximum(
                macc_ref[pl.ds(i * _BLK, _BLK), :], _rowmax(dik))
            cm = jnp.max(dik, axis=0, keepdims=True).reshape(_BLK, 1)
            macc_ref[kk * _BLK:(kk + 1) * _BLK, :] = jnp.maximum(
                macc_ref[kk * _BLK:(kk + 1) * _BLK, :], cm)

    @pl.when(i == _R - 1)
    def _fin():
        acc_ref[0, 0] = _logdist_sum(macc_ref[...])


def kernel(student_output):
    acc = pl.pallas_call(
        _koleo_body,
        grid=(_R,),
        in_specs=[pl.BlockSpec((_BLK, _D), lambda i: (i, 0))],
        out_specs=pl.BlockSpec(
            block_shape=(1, 1),
            index_map=lambda i: (0, 0),
            memory_space=pltpu.SMEM,
        ),
        out_shape=jax.ShapeDtypeStruct((1, 1), jnp.float32),
        scratch_shapes=[
            pltpu.VMEM((_B, _D), jnp.bfloat16),
            pltpu.VMEM((_B, 1), jnp.float32),
        ],
        compiler_params=pltpu.CompilerParams(
            dimension_semantics=("arbitrary",)),
    )(student_output)
    return -(acc[0, 0] / _B)


# speedup vs baseline: 1.2917x; 1.2917x over previous
import jax
import jax.numpy as jnp
from jax.experimental import pallas as pl
from jax.experimental.pallas import tpu as pltpu

_EPS = 1e-8
_B = 4096
_D = 256
_BLK = 2048
_R = _B // _BLK


def _normalize(x):
    nrm2 = jax.lax.dot_general(
        (x * x).astype(jnp.bfloat16), jnp.ones((_D, 1), jnp.bfloat16),
        (((1,), (0,)), ((), ())),
        preferred_element_type=jnp.float32)
    inv = 1.0 / jnp.maximum(jnp.sqrt(nrm2), _EPS)
    return (x * inv).astype(jnp.bfloat16)


def _dot_nt(a, b):
    return jax.lax.dot_general(
        a, b, (((1,), (1,)), ((), ())),
        preferred_element_type=jnp.float32)


def _mask_diag(d):
    diag = (jax.lax.broadcasted_iota(jnp.int32, (_BLK, _BLK), 0)
            == jax.lax.broadcasted_iota(jnp.int32, (_BLK, _BLK), 1))
    return jnp.where(diag, jnp.float32(-1.0), d)


def _rowmax(d):
    parts = [d[:, c * 128:(c + 1) * 128] for c in range(_BLK // 128)]
    while len(parts) > 1:
        parts = [jnp.maximum(parts[p], parts[p + 1])
                 for p in range(0, len(parts), 2)]
    return jnp.max(parts[0], axis=1, keepdims=True)


def _logdist_sum(m):
    dist2 = 2.0 + _D * _EPS * _EPS - 2.0 * m
    dist = jnp.sqrt(jnp.maximum(dist2, 0.0))
    return jnp.sum(jnp.log(dist + _EPS))


def _koleo_body(x_ref, acc_ref, y0_ref, rmax_ref):
    i = pl.program_id(0)

    @pl.when(i == 0)
    def _first_half():
        xn0 = _normalize(x_ref[...])
        y0_ref[...] = xn0
        d00 = _mask_diag(_dot_nt(xn0, xn0))
        rmax_ref[...] = _rowmax(d00)

    @pl.when(i == 1)
    def _second_half():
        xn1 = _normalize(x_ref[...])
        d10 = _dot_nt(xn1, y0_ref[...])
        c0 = jnp.max(d10, axis=0, keepdims=True)
        m0 = jnp.maximum(rmax_ref[...], c0.reshape(_BLK, 1))
        s0 = _logdist_sum(m0)
        d11 = _mask_diag(_dot_nt(xn1, xn1))
        m1 = jnp.maximum(_rowmax(d10), _rowmax(d11))
        acc_ref[0, 0] = s0 + _logdist_sum(m1)


def kernel(student_output):
    acc = pl.pallas_call(
        _koleo_body,
        grid=(_R,),
        in_specs=[pl.BlockSpec((_BLK, _D), lambda i: (i, 0))],
        out_specs=pl.BlockSpec(
            block_shape=(1, 1),
            index_map=lambda i: (0, 0),
            memory_space=pltpu.SMEM,
        ),
        out_shape=jax.ShapeDtypeStruct((1, 1), jnp.float32),
        scratch_shapes=[
            pltpu.VMEM((_BLK, _D), jnp.bfloat16),
            pltpu.VMEM((_BLK, 1), jnp.float32),
        ],
        compiler_params=pltpu.CompilerParams(
            dimension_semantics=("arbitrary",)),
    )(student_output)
    return -(acc[0, 0] / _B)


# speedup vs baseline: 1.4216x; 1.1006x over previous
import jax
import jax.numpy as jnp
from jax.experimental import pallas as pl
from jax.experimental.pallas import tpu as pltpu

_EPS = 1e-8
_B = 4096
_D = 256
_BLK = 2048
_R = _B // _BLK


def _normalize(x):
    nrm2 = jax.lax.dot_general(
        (x * x).astype(jnp.bfloat16), jnp.ones((_D, 1), jnp.bfloat16),
        (((1,), (0,)), ((), ())),
        preferred_element_type=jnp.float32)
    inv = 1.0 / jnp.maximum(jnp.sqrt(nrm2), _EPS)
    return (x * inv).astype(jnp.bfloat16)


def _dot_nt(a, b):
    return jax.lax.dot_general(
        a, b, (((1,), (1,)), ((), ())),
        preferred_element_type=jnp.float32)


def _mask_diag(d):
    diag = (jax.lax.broadcasted_iota(jnp.int32, (_BLK, _BLK), 0)
            == jax.lax.broadcasted_iota(jnp.int32, (_BLK, _BLK), 1))
    return jnp.where(diag, jnp.float32(-1.0), d)


def _rowmax(d):
    parts = [d[:, c * 128:(c + 1) * 128] for c in range(_BLK // 128)]
    while len(parts) > 1:
        parts = [jnp.maximum(parts[p], parts[p + 1])
                 for p in range(0, len(parts), 2)]
    return jnp.max(parts[0], axis=1, keepdims=True)


def _logdist_sum(m):
    dist2 = 2.0 + _D * _EPS * _EPS - 2.0 * m
    dist = jnp.sqrt(jnp.maximum(dist2, 0.0))
    return jnp.sum(jnp.log(dist + _EPS))


def _koleo_body(x_ref, acc_ref, y0_ref, rmax_ref):
    i = pl.program_id(0)

    @pl.when(i == 0)
    def _first_half():
        xn0 = _normalize(x_ref[...])
        y0_ref[...] = xn0
        d00 = _mask_diag(_dot_nt(xn0, xn0))
        rmax_ref[...] = _rowmax(d00)

    @pl.when(i == 1)
    def _second_half():
        xn1 = _normalize(x_ref[...])
        d10 = _dot_nt(xn1, y0_ref[...])
        c0 = jnp.max(d10, axis=0, keepdims=True)
        m0 = jnp.maximum(rmax_ref[...], c0.reshape(_BLK, 1))
        s0 = _logdist_sum(m0)
        d11 = _mask_diag(_dot_nt(xn1, xn1))
        m1 = jnp.maximum(_rowmax(d10), _rowmax(d11))
        acc_ref[0, 0] = -(s0 + _logdist_sum(m1)) * jnp.float32(1.0 / _B)


def kernel(student_output):
    acc = pl.pallas_call(
        _koleo_body,
        grid=(_R,),
        in_specs=[pl.BlockSpec((_BLK, _D), lambda i: (i, 0))],
        out_specs=pl.BlockSpec(
            block_shape=(1, 1),
            index_map=lambda i: (0, 0),
            memory_space=pltpu.SMEM,
        ),
        out_shape=jax.ShapeDtypeStruct((1, 1), jnp.float32),
        scratch_shapes=[
            pltpu.VMEM((_BLK, _D), jnp.bfloat16),
            pltpu.VMEM((_BLK, 1), jnp.float32),
        ],
        compiler_params=pltpu.CompilerParams(
            dimension_semantics=("arbitrary",)),
    )(student_output)
    return acc[0, 0]
